# baseline (speedup 1.0000x reference)
import jax
import jax.numpy as jnp
from jax.experimental import pallas as pl
from jax.experimental.pallas import tpu as pltpu

_B, _D, _E, _H, _C = 2048, 1024, 16, 256, 100
_CP = 128
_TB = 512


def _fused_kernel(x_ref, proto_ref, W1_ref, b1_ref, g_ref, bb_ref,
                  W2_ref, b2_ref, out_ref, w_scratch):
    e = pl.program_id(1)
    xb = x_ref[...]

    @pl.when(e == 0)
    def _compute_router():
        xn = xb / (jnp.sqrt(jnp.sum(xb * xb, axis=1, keepdims=True)) + 1e-8)
        p = proto_ref[...]
        pn = p / (jnp.sqrt(jnp.sum(p * p, axis=1, keepdims=True)) + 1e-8)
        sims = jnp.dot(xn, pn.T, preferred_element_type=jnp.float32)
        w_scratch[...] = jax.nn.softmax(sims, axis=-1)

    h = jnp.dot(xb, W1_ref[0], preferred_element_type=jnp.float32) + b1_ref[0]
    mu = jnp.mean(h, axis=-1, keepdims=True)
    var = jnp.mean((h - mu) ** 2, axis=-1, keepdims=True)
    h = (h - mu) / jnp.sqrt(var + 1e-5)
    h = h * g_ref[0] + bb_ref[0]
    h = jax.nn.gelu(h)
    logits = jnp.dot(h, W2_ref[0], preferred_element_type=jnp.float32) + b2_ref[0]

    w = w_scratch[...]
    lane = jax.lax.broadcasted_iota(jnp.int32, w.shape, 1)
    w_col = jnp.sum(jnp.where(lane == e, w, 0.0), axis=1, keepdims=True)

    @pl.when(e == 0)
    def _init():
        out_ref[...] = w_col * logits

    @pl.when(e != 0)
    def _acc():
        out_ref[...] += w_col * logits


@jax.jit
def kernel(x, prototypes, W1, b1, ln_g, ln_b, W2, b2):
    W2p = jnp.pad(W2, ((0, 0), (0, 0), (0, _CP - _C)))
    b2p = jnp.pad(b2, ((0, 0), (0, _CP - _C)))
    b1r = b1.reshape(_E, 1, _H)
    gr = ln_g.reshape(_E, 1, _H)
    br = ln_b.reshape(_E, 1, _H)
    b2r = b2p.reshape(_E, 1, _CP)
    nb = _B // _TB
    out = pl.pallas_call(
        _fused_kernel,
        grid=(nb, _E),
        in_specs=[
            pl.BlockSpec((_TB, _D), lambda b, e: (b, 0)),
            pl.BlockSpec((_E, _D), lambda b, e: (0, 0)),
            pl.BlockSpec((1, _D, _H), lambda b, e: (e, 0, 0)),
            pl.BlockSpec((1, 1, _H), lambda b, e: (e, 0, 0)),
            pl.BlockSpec((1, 1, _H), lambda b, e: (e, 0, 0)),
            pl.BlockSpec((1, 1, _H), lambda b, e: (e, 0, 0)),
            pl.BlockSpec((1, _H, _CP), lambda b, e: (e, 0, 0)),
            pl.BlockSpec((1, 1, _CP), lambda b, e: (e, 0, 0)),
        ],
        out_specs=pl.BlockSpec((_TB, _CP), lambda b, e: (b, 0)),
        out_shape=jax.ShapeDtypeStruct((_B, _CP), jnp.float32),
        scratch_shapes=[pltpu.VMEM((_TB, _E), jnp.float32)],
        compiler_params=pltpu.CompilerParams(
            dimension_semantics=("parallel", "arbitrary")),
    )(x, prototypes, W1, b1r, gr, br, W2p, b2r)
    return out[:, :_C]
